# Initial kernel scaffold; baseline (speedup 1.0000x reference)
import jax
import jax.numpy as jnp
from jax.experimental import pallas as pl


def kernel(input_ids, word_table, pos_table, ln_gamma, ln_beta):
    raise NotImplementedError("write your pallas kernel here")


# speedup vs baseline: 1.3722x; 1.3722x over previous
import functools

import jax
import jax.numpy as jnp
from jax import lax
from jax.experimental import pallas as pl
from jax.experimental.pallas import tpu as pltpu
from jax.experimental.pallas import tpu_sc as plsc

EPS = 1e-12

_NUM_CORES = 2
_NUM_SUBCORES = 16
_NW = _NUM_CORES * _NUM_SUBCORES

_CHUNK = 64


def _sc_gather(table, idx):
    tok, h = idx.shape[0], table.shape[1]
    assert tok % (_NW * _CHUNK) == 0
    per_w = tok // _NW
    n_chunks = per_w // _CHUNK

    mesh = plsc.VectorSubcoreMesh(
        core_axis_name="c", subcore_axis_name="s",
        num_cores=_NUM_CORES, num_subcores=_NUM_SUBCORES)

    @functools.partial(
        pl.kernel,
        mesh=mesh,
        out_type=jax.ShapeDtypeStruct((tok, h), jnp.float32),
        scratch_types=[
            pltpu.VMEM((per_w,), jnp.int32),
            pltpu.VMEM((_CHUNK, h), jnp.float32),
            pltpu.VMEM((_CHUNK, h), jnp.float32),
            pltpu.SemaphoreType.DMA,
            pltpu.SemaphoreType.DMA,
        ],
    )
    def gather_kernel(table_hbm, idx_hbm, out_hbm, idx_v, buf0, buf1, sem0, sem1):
        wid = lax.axis_index("s") * _NUM_CORES + lax.axis_index("c")
        base = wid * per_w
        pltpu.sync_copy(idx_hbm.at[pl.ds(base, per_w)], idx_v)

        bufs = (buf0, buf1)
        sems = (sem0, sem1)
        copies = [None, None]
        copies[0] = pltpu.async_copy(
            table_hbm.at[idx_v.at[pl.ds(0, _CHUNK)]], bufs[0], sems[0])
        for c in range(1, n_chunks):
            copies[c % 2] = pltpu.async_copy(
                table_hbm.at[idx_v.at[pl.ds(c * _CHUNK, _CHUNK)]],
                bufs[c % 2], sems[c % 2])
            copies[(c - 1) % 2].wait()
            pltpu.sync_copy(
                bufs[(c - 1) % 2],
                out_hbm.at[pl.ds(base + (c - 1) * _CHUNK, _CHUNK)])
        copies[(n_chunks - 1) % 2].wait()
        pltpu.sync_copy(
            bufs[(n_chunks - 1) % 2],
            out_hbm.at[pl.ds(base + (n_chunks - 1) * _CHUNK, _CHUNK)])

    return gather_kernel(table, idx)


def _add_ln_tc(x, pos, gamma, beta):
    b, s, h = x.shape
    bs = 512
    grid = (b, s // bs)

    def body(x_ref, pos_ref, g_ref, b_ref, o_ref):
        xv = x_ref[0] + pos_ref[...]
        mean = jnp.mean(xv, axis=-1, keepdims=True)
        xc = xv - mean
        var = jnp.mean(xc * xc, axis=-1, keepdims=True)
        inv = lax.rsqrt(var + EPS)
        o_ref[0] = (xc * inv) * g_ref[...] + b_ref[...]

    return pl.pallas_call(
        body,
        grid=grid,
        in_specs=[
            pl.BlockSpec((1, bs, h), lambda i, j: (i, j, 0)),
            pl.BlockSpec((bs, h), lambda i, j: (j, 0)),
            pl.BlockSpec((1, h), lambda i, j: (0, 0)),
            pl.BlockSpec((1, h), lambda i, j: (0, 0)),
        ],
        out_specs=pl.BlockSpec((1, bs, h), lambda i, j: (i, j, 0)),
        out_shape=jax.ShapeDtypeStruct((b, s, h), jnp.float32),
    )(x, pos, gamma, beta)


def kernel(input_ids, word_table, pos_table, ln_gamma, ln_beta):
    b, s = input_ids.shape
    h = word_table.shape[1]
    idx = input_ids.reshape(-1).astype(jnp.int32)
    gathered = _sc_gather(word_table, idx)
    x = gathered.reshape(b, s, h)
    return _add_ln_tc(x, pos_table[:s], ln_gamma.reshape(1, h),
                      ln_beta.reshape(1, h))


# speedup vs baseline: 1.4294x; 1.0417x over previous
import functools

import jax
import jax.numpy as jnp
from jax import lax
from jax.experimental import pallas as pl
from jax.experimental.pallas import tpu as pltpu
from jax.experimental.pallas import tpu_sc as plsc

EPS = 1e-12

_NUM_CORES = 2
_NUM_SUBCORES = 16
_NW = _NUM_CORES * _NUM_SUBCORES

_CHUNK = 64


def _sc_gather(table, idx):
    tok, h = idx.shape[0], table.shape[1]
    assert tok % (_NW * _CHUNK) == 0
    per_w = tok // _NW
    n_chunks = per_w // _CHUNK

    mesh = plsc.VectorSubcoreMesh(
        core_axis_name="c", subcore_axis_name="s",
        num_cores=_NUM_CORES, num_subcores=_NUM_SUBCORES)

    @functools.partial(
        pl.kernel,
        mesh=mesh,
        out_type=jax.ShapeDtypeStruct((tok, h), jnp.float32),
        scratch_types=[
            pltpu.VMEM((per_w,), jnp.int32),
            pltpu.VMEM((_CHUNK, h), jnp.float32),
            pltpu.VMEM((_CHUNK, h), jnp.float32),
            pltpu.SemaphoreType.DMA,
            pltpu.SemaphoreType.DMA,
        ],
    )
    def gather_kernel(table_hbm, idx_hbm, out_hbm, idx_v, buf0, buf1, sem0, sem1):
        wid = lax.axis_index("s") * _NUM_CORES + lax.axis_index("c")
        base = wid * per_w
        pltpu.sync_copy(idx_hbm.at[pl.ds(base, per_w)], idx_v)

        bufs = (buf0, buf1)
        sems = (sem0, sem1)
        copies = [None, None]
        copies[0] = pltpu.async_copy(
            table_hbm.at[idx_v.at[pl.ds(0, _CHUNK)]], bufs[0], sems[0])
        for c in range(1, n_chunks):
            copies[c % 2] = pltpu.async_copy(
                table_hbm.at[idx_v.at[pl.ds(c * _CHUNK, _CHUNK)]],
                bufs[c % 2], sems[c % 2])
            copies[(c - 1) % 2].wait()
            pltpu.sync_copy(
                bufs[(c - 1) % 2],
                out_hbm.at[pl.ds(base + (c - 1) * _CHUNK, _CHUNK)])
        copies[(n_chunks - 1) % 2].wait()
        pltpu.sync_copy(
            bufs[(n_chunks - 1) % 2],
            out_hbm.at[pl.ds(base + (n_chunks - 1) * _CHUNK, _CHUNK)])

    return gather_kernel(table, idx)


def _add_ln_tc(x, pos, gamma, beta):
    b, s, h = x.shape
    bs = 512
    grid = (s // bs, b)

    def body(x_ref, pos_ref, g_ref, b_ref, o_ref):
        xv = x_ref[0] + pos_ref[...]
        mean = jnp.mean(xv, axis=-1, keepdims=True)
        xc = xv - mean
        var = jnp.mean(xc * xc, axis=-1, keepdims=True)
        inv = lax.rsqrt(var + EPS)
        o_ref[0] = (xc * inv) * g_ref[...] + b_ref[...]

    return pl.pallas_call(
        body,
        grid=grid,
        in_specs=[
            pl.BlockSpec((1, bs, h), lambda i, j: (j, i, 0)),
            pl.BlockSpec((bs, h), lambda i, j: (i, 0)),
            pl.BlockSpec((1, h), lambda i, j: (0, 0)),
            pl.BlockSpec((1, h), lambda i, j: (0, 0)),
        ],
        out_specs=pl.BlockSpec((1, bs, h), lambda i, j: (j, i, 0)),
        out_shape=jax.ShapeDtypeStruct((b, s, h), jnp.float32),
    )(x, pos, gamma, beta)


def kernel(input_ids, word_table, pos_table, ln_gamma, ln_beta):
    b, s = input_ids.shape
    h = word_table.shape[1]
    idx = input_ids.reshape(-1).astype(jnp.int32)
    gathered = _sc_gather(word_table, idx)
    x = gathered.reshape(b, s, h)
    return _add_ln_tc(x, pos_table[:s], ln_gamma.reshape(1, h),
                      ln_beta.reshape(1, h))
